# baseline (speedup 1.0000x reference)
import functools

import jax
import jax.numpy as jnp
from jax import lax
from jax.experimental import pallas as pl
from jax.experimental.pallas import tpu as pltpu
from jax.experimental.pallas import tpu_sc as plsc

D_FEAT = 128
LANES = 16
N_CORES = 2
N_SUBCORES = 16
N_WORKERS = N_CORES * N_SUBCORES
CHUNK = 192
GROUPS = CHUNK // LANES
D_WORDS = D_FEAT // 2
WPF = D_WORDS // LANES
NBUF = 2


def _make_kernel(e_pad, n_pad):
  ew = e_pad // N_WORKERS
  n_chunks = ew // CHUNK
  assert n_chunks % NBUF == 0
  assert n_pad % (8 * N_SUBCORES) == 0
  rows_per_sub = n_pad // N_SUBCORES
  mesh = plsc.VectorSubcoreMesh(core_axis_name="c", subcore_axis_name="s")

  @functools.partial(
      pl.kernel,
      mesh=mesh,
      compiler_params=pltpu.CompilerParams(needs_layout_passes=False,
                                           use_tc_tiling_on_sc=False),
      out_type=jax.ShapeDtypeStruct((e_pad,), jnp.float32),
      scratch_types=[
          pltpu.VMEM_SHARED((n_pad, D_WORDS), jnp.float32),
          pltpu.VMEM((2 * ew,), jnp.int32),
          pltpu.VMEM((ew,), jnp.float32),
      ] + [pltpu.VMEM((2 * CHUNK, D_WORDS), jnp.float32)] * NBUF
        + [pltpu.SemaphoreType.DMA] * NBUF,
  )
  def score_kernel(h_hbm, idx_hbm, out_hbm, h_sh, idx_all, out_all, *rest):
    rows = rest[:NBUF]
    sems = rest[NBUF:]

    cid = lax.axis_index("c")
    sid = lax.axis_index("s")
    wid = sid * N_CORES + cid
    base = wid * ew
    chunk0 = wid * n_chunks
    lane = lax.iota(jnp.int32, LANES)
    rots = [jnp.bitwise_and(lane + r, LANES - 1) for r in (8, 4, 2, 1)]
    places = [jnp.bitwise_and(lane - k, LANES - 1) for k in range(LANES)]

    def rot(x, perm):
      return x.at[perm].get(mode="promise_in_bounds")

    pltpu.sync_copy(h_hbm.at[pl.ds(sid * rows_per_sub, rows_per_sub)],
                    h_sh.at[pl.ds(sid * rows_per_sub, rows_per_sub)])
    pltpu.sync_copy(idx_hbm.at[pl.ds(chunk0 * 2 * CHUNK, 2 * ew)], idx_all)
    plsc.subcore_barrier()

    def fire(ch, b):
      ii = idx_all.at[pl.ds(ch * 2 * CHUNK, 2 * CHUNK)]
      pltpu.async_copy(h_sh.at[ii], rows[b], sems[b])

    def wait_gather(ch, b):
      ii = idx_all.at[pl.ds(ch * 2 * CHUNK, 2 * CHUNK)]
      pltpu.make_async_copy(h_sh.at[ii], rows[b], sems[b]).wait()

    for b in range(NBUF):
      fire(b, b)

    def loop_body(j, carry):
      for b in range(NBUF):
        ch = NBUF * j + b
        wait_gather(ch, b)

        def group_body(g, carry2, b=b):
          acc = jnp.zeros((LANES,), jnp.float32)
          for k in range(LANES):
            e = g * LANES + k
            ps = []
            for i in range(WPF):
              uw = plsc.bitcast(rows[b][e, pl.ds(i * LANES, LANES)],
                                jnp.bfloat16)
              vw = plsc.bitcast(rows[b][CHUNK + e, pl.ds(i * LANES, LANES)],
                                jnp.bfloat16)
              lo, hi = plsc.unpack(uw * vw, format=plsc.PackFormat.INTERLEAVED)
              ps.append(lo + hi)
            m = (ps[0] + ps[1]) + (ps[2] + ps[3])
            for p in rots:
              m = m + rot(m, p)
            t = m if k == 0 else rot(m, places[k])
            acc = jnp.where(lane == k, t, acc)
          out_all[pl.ds(ch * CHUNK + g * LANES, LANES)] = acc
          return carry2

        lax.fori_loop(0, GROUPS, group_body, 0)
        fire(jnp.minimum(ch + NBUF, n_chunks - 1), b)
      return carry

    lax.fori_loop(0, n_chunks // NBUF, loop_body, 0)
    for b in range(NBUF):
      wait_gather(0, b)
    pltpu.sync_copy(out_all, out_hbm.at[pl.ds(base, ew)])

  return score_kernel


def kernel(h, edge_index):
  e = edge_index.shape[1]
  epc = N_WORKERS * CHUNK * NBUF
  e_pad = ((e + epc - 1) // epc) * epc
  src = edge_index[0].astype(jnp.int32)
  dst = edge_index[1].astype(jnp.int32)
  if e_pad != e:
    src = jnp.pad(src, (0, e_pad - e))
    dst = jnp.pad(dst, (0, e_pad - e))
  idx = jnp.stack([src.reshape(-1, CHUNK), dst.reshape(-1, CHUNK)],
                  axis=1).reshape(-1)
  npc = 8 * N_SUBCORES
  n_pad = ((h.shape[0] + npc - 1) // npc) * npc
  if n_pad != h.shape[0]:
    h = jnp.pad(h, ((0, n_pad - h.shape[0]), (0, 0)))
  hw = jax.lax.bitcast_convert_type(
      h.astype(jnp.bfloat16).reshape(n_pad, D_FEAT // 2, 2), jnp.float32)
  out = _make_kernel(e_pad, n_pad)(hw, idx)
  return out[:e, None]


# speedup vs baseline: 1.0991x; 1.0991x over previous
import functools

import jax
import jax.numpy as jnp
from jax import lax
from jax.experimental import pallas as pl
from jax.experimental.pallas import tpu as pltpu
from jax.experimental.pallas import tpu_sc as plsc

D_FEAT = 128
LANES = 16
N_CORES = 2
N_SUBCORES = 16
N_WORKERS = N_CORES * N_SUBCORES
CHUNK = 128
GROUPS = CHUNK // LANES
D_WORDS = D_FEAT // 2
WPF = D_WORDS // LANES
NBUF = 3


def _make_kernel(e_pad, n_pad):
  ew = e_pad // N_WORKERS
  n_chunks = ew // CHUNK
  assert n_chunks % NBUF == 0
  assert n_pad % (8 * N_SUBCORES) == 0
  rows_per_sub = n_pad // N_SUBCORES
  mesh = plsc.VectorSubcoreMesh(core_axis_name="c", subcore_axis_name="s")

  @functools.partial(
      pl.kernel,
      mesh=mesh,
      compiler_params=pltpu.CompilerParams(needs_layout_passes=False,
                                           use_tc_tiling_on_sc=False),
      out_type=jax.ShapeDtypeStruct((e_pad,), jnp.float32),
      scratch_types=[
          pltpu.VMEM_SHARED((n_pad, D_WORDS), jnp.float32),
          pltpu.VMEM((2 * ew,), jnp.int32),
          pltpu.VMEM((ew,), jnp.float32),
      ] + [pltpu.VMEM((2 * CHUNK, D_WORDS), jnp.float32)] * NBUF
        + [pltpu.SemaphoreType.DMA] * NBUF,
  )
  def score_kernel(h_hbm, idx_hbm, out_hbm, h_sh, idx_all, out_all, *rest):
    rows = rest[:NBUF]
    sems = rest[NBUF:]

    cid = lax.axis_index("c")
    sid = lax.axis_index("s")
    wid = sid * N_CORES + cid
    base = wid * ew
    chunk0 = wid * n_chunks
    lane = lax.iota(jnp.int32, LANES)
    rots = [jnp.bitwise_and(lane + r, LANES - 1) for r in (8, 4, 2, 1)]
    places = [jnp.bitwise_and(lane - k, LANES - 1) for k in range(LANES)]

    def rot(x, perm):
      return x.at[perm].get(mode="promise_in_bounds")

    pltpu.sync_copy(h_hbm.at[pl.ds(sid * rows_per_sub, rows_per_sub)],
                    h_sh.at[pl.ds(sid * rows_per_sub, rows_per_sub)])
    pltpu.sync_copy(idx_hbm.at[pl.ds(chunk0 * 2 * CHUNK, 2 * ew)], idx_all)
    plsc.subcore_barrier()

    def fire(ch, b):
      ii = idx_all.at[pl.ds(ch * 2 * CHUNK, 2 * CHUNK)]
      pltpu.async_copy(h_sh.at[ii], rows[b], sems[b])

    def wait_gather(ch, b):
      ii = idx_all.at[pl.ds(ch * 2 * CHUNK, 2 * CHUNK)]
      pltpu.make_async_copy(h_sh.at[ii], rows[b], sems[b]).wait()

    for b in range(NBUF):
      fire(b, b)

    def loop_body(j, carry):
      for b in range(NBUF):
        ch = NBUF * j + b
        wait_gather(ch, b)

        def group_body(g, carry2, b=b):
          acc = jnp.zeros((LANES,), jnp.float32)
          for k in range(LANES):
            e = g * LANES + k
            ps = []
            for i in range(WPF):
              uw = plsc.bitcast(rows[b][e, pl.ds(i * LANES, LANES)],
                                jnp.bfloat16)
              vw = plsc.bitcast(rows[b][CHUNK + e, pl.ds(i * LANES, LANES)],
                                jnp.bfloat16)
              lo, hi = plsc.unpack(uw * vw, format=plsc.PackFormat.INTERLEAVED)
              ps.append(lo + hi)
            m = (ps[0] + ps[1]) + (ps[2] + ps[3])
            for p in rots:
              m = m + rot(m, p)
            t = m if k == 0 else rot(m, places[k])
            acc = jnp.where(lane == k, t, acc)
          out_all[pl.ds(ch * CHUNK + g * LANES, LANES)] = acc
          return carry2

        lax.fori_loop(0, GROUPS, group_body, 0)
        fire(jnp.minimum(ch + NBUF, n_chunks - 1), b)
      return carry

    lax.fori_loop(0, n_chunks // NBUF, loop_body, 0)
    for b in range(NBUF):
      wait_gather(0, b)
    pltpu.sync_copy(out_all, out_hbm.at[pl.ds(base, ew)])

  return score_kernel


def kernel(h, edge_index):
  e = edge_index.shape[1]
  epc = N_WORKERS * CHUNK * NBUF
  e_pad = ((e + epc - 1) // epc) * epc
  src = edge_index[0].astype(jnp.int32)
  dst = edge_index[1].astype(jnp.int32)
  if e_pad != e:
    src = jnp.pad(src, (0, e_pad - e))
    dst = jnp.pad(dst, (0, e_pad - e))
  idx = jnp.stack([src.reshape(-1, CHUNK), dst.reshape(-1, CHUNK)],
                  axis=1).reshape(-1)
  npc = 8 * N_SUBCORES
  n_pad = ((h.shape[0] + npc - 1) // npc) * npc
  if n_pad != h.shape[0]:
    h = jnp.pad(h, ((0, n_pad - h.shape[0]), (0, 0)))
  hw = jax.lax.bitcast_convert_type(
      h.astype(jnp.bfloat16).reshape(n_pad, D_FEAT // 2, 2), jnp.float32)
  out = _make_kernel(e_pad, n_pad)(hw, idx)
  return out[:e, None]


# speedup vs baseline: 1.1315x; 1.0294x over previous
import functools

import jax
import jax.numpy as jnp
from jax import lax
from jax.experimental import pallas as pl
from jax.experimental.pallas import tpu as pltpu
from jax.experimental.pallas import tpu_sc as plsc

D_FEAT = 128
LANES = 16
N_CORES = 2
N_SUBCORES = 16
N_WORKERS = N_CORES * N_SUBCORES
CHUNK = 128
GROUPS = CHUNK // LANES
D_WORDS = D_FEAT // 2
WPF = D_WORDS // LANES
NBUF = 2


def _make_kernel(e_pad, n_pad):
  ew = e_pad // N_WORKERS
  n_chunks = ew // CHUNK
  assert n_chunks % NBUF == 0
  assert n_pad % (8 * N_SUBCORES) == 0
  rows_per_sub = n_pad // N_SUBCORES
  mesh = plsc.VectorSubcoreMesh(core_axis_name="c", subcore_axis_name="s")

  @functools.partial(
      pl.kernel,
      mesh=mesh,
      compiler_params=pltpu.CompilerParams(needs_layout_passes=False,
                                           use_tc_tiling_on_sc=False),
      out_type=jax.ShapeDtypeStruct((e_pad,), jnp.float32),
      scratch_types=[
          pltpu.VMEM_SHARED((n_pad, D_WORDS), jnp.float32),
          pltpu.VMEM((2 * ew,), jnp.int32),
          pltpu.VMEM((ew,), jnp.float32),
      ] + [pltpu.VMEM((2 * CHUNK, D_WORDS), jnp.float32)] * NBUF
        + [pltpu.SemaphoreType.DMA] * NBUF,
  )
  def score_kernel(h_hbm, idx_hbm, out_hbm, h_sh, idx_all, out_all, *rest):
    rows = rest[:NBUF]
    sems = rest[NBUF:]

    cid = lax.axis_index("c")
    sid = lax.axis_index("s")
    wid = sid * N_CORES + cid
    base = wid * ew
    chunk0 = wid * n_chunks
    lane = lax.iota(jnp.int32, LANES)
    rots = [jnp.bitwise_and(lane + r, LANES - 1) for r in (8, 4, 2, 1)]
    places = [jnp.bitwise_and(lane - k, LANES - 1) for k in range(LANES)]

    def rot(x, perm):
      return x.at[perm].get(mode="promise_in_bounds")

    pltpu.sync_copy(h_hbm.at[pl.ds(sid * rows_per_sub, rows_per_sub)],
                    h_sh.at[pl.ds(sid * rows_per_sub, rows_per_sub)])
    pltpu.sync_copy(idx_hbm.at[pl.ds(chunk0 * 2 * CHUNK, 2 * ew)], idx_all)
    plsc.subcore_barrier()

    def fire(ch, b):
      ii = idx_all.at[pl.ds(ch * 2 * CHUNK, 2 * CHUNK)]
      pltpu.async_copy(h_sh.at[ii], rows[b], sems[b])

    def wait_gather(ch, b):
      ii = idx_all.at[pl.ds(ch * 2 * CHUNK, 2 * CHUNK)]
      pltpu.make_async_copy(h_sh.at[ii], rows[b], sems[b]).wait()

    for b in range(NBUF):
      fire(b, b)

    def loop_body(j, carry):
      for b in range(NBUF):
        ch = NBUF * j + b
        wait_gather(ch, b)

        def group_body(g, carry2, b=b):
          acc = jnp.zeros((LANES,), jnp.float32)
          for k in range(LANES):
            e = g * LANES + k
            ps = []
            for i in range(WPF):
              uw = plsc.bitcast(rows[b][e, pl.ds(i * LANES, LANES)],
                                jnp.bfloat16)
              vw = plsc.bitcast(rows[b][CHUNK + e, pl.ds(i * LANES, LANES)],
                                jnp.bfloat16)
              lo, hi = plsc.unpack(uw * vw, format=plsc.PackFormat.INTERLEAVED)
              ps.append(lo + hi)
            m = (ps[0] + ps[1]) + (ps[2] + ps[3])
            for p in rots:
              m = m + rot(m, p)
            t = m if k == 0 else rot(m, places[k])
            acc = jnp.where(lane == k, t, acc)
          out_all[pl.ds(ch * CHUNK + g * LANES, LANES)] = acc
          return carry2

        lax.fori_loop(0, GROUPS, group_body, 0)
        fire(jnp.minimum(ch + NBUF, n_chunks - 1), b)
      return carry

    lax.fori_loop(0, n_chunks // NBUF, loop_body, 0)
    for b in range(NBUF):
      wait_gather(0, b)
    pltpu.sync_copy(out_all, out_hbm.at[pl.ds(base, ew)])

  return score_kernel


def kernel(h, edge_index):
  e = edge_index.shape[1]
  epc = N_WORKERS * CHUNK * NBUF
  e_pad = ((e + epc - 1) // epc) * epc
  src = edge_index[0].astype(jnp.int32)
  dst = edge_index[1].astype(jnp.int32)
  if e_pad != e:
    src = jnp.pad(src, (0, e_pad - e))
    dst = jnp.pad(dst, (0, e_pad - e))
  idx = jnp.stack([src.reshape(-1, CHUNK), dst.reshape(-1, CHUNK)],
                  axis=1).reshape(-1)
  npc = 8 * N_SUBCORES
  n_pad = ((h.shape[0] + npc - 1) // npc) * npc
  if n_pad != h.shape[0]:
    h = jnp.pad(h, ((0, n_pad - h.shape[0]), (0, 0)))
  hw = jax.lax.bitcast_convert_type(
      h.astype(jnp.bfloat16).reshape(n_pad, D_FEAT // 2, 2), jnp.float32)
  out = _make_kernel(e_pad, n_pad)(hw, idx)
  return out[:e, None]


# speedup vs baseline: 1.2871x; 1.1375x over previous
import functools

import jax
import jax.numpy as jnp
from jax import lax
from jax.experimental import pallas as pl
from jax.experimental.pallas import tpu as pltpu
from jax.experimental.pallas import tpu_sc as plsc

D_FEAT = 128
LANES = 16
N_CORES = 2
N_SUBCORES = 16
N_WORKERS = N_CORES * N_SUBCORES
CHUNK = 80
GROUPS = CHUNK // LANES
D_WORDS = D_FEAT // 2
WPF = D_WORDS // LANES
NBUF = 2
STAGE_ROWS = 640
STAGE_BLK = 32


def _make_kernel(e, n_nodes):
  ew = e // N_WORKERS
  n_chunks = ew // CHUNK
  assert ew % CHUNK == 0 and n_nodes % 8 == 0 and n_nodes >= STAGE_ROWS
  mesh = plsc.VectorSubcoreMesh(core_axis_name="c", subcore_axis_name="s")

  @functools.partial(
      pl.kernel,
      mesh=mesh,
      compiler_params=pltpu.CompilerParams(needs_layout_passes=False,
                                           use_tc_tiling_on_sc=False),
      out_type=jax.ShapeDtypeStruct((e,), jnp.float32),
      scratch_types=[
          pltpu.VMEM_SHARED((n_nodes, D_WORDS), jnp.float32),
          pltpu.VMEM((ew,), jnp.int32),
          pltpu.VMEM((ew,), jnp.int32),
          pltpu.VMEM((ew,), jnp.float32),
          pltpu.VMEM((STAGE_BLK, D_FEAT), jnp.float32),
          pltpu.VMEM((STAGE_BLK, D_WORDS), jnp.float32),
      ] + [pltpu.VMEM((2 * CHUNK, D_WORDS), jnp.float32)] * NBUF
        + [pltpu.SemaphoreType.DMA] * (2 * NBUF),
  )
  def score_kernel(h_hbm, src_hbm, dst_hbm, out_hbm, h_sh,
                   src_all, dst_all, out_all, fbuf, pbuf, *rest):
    rows = rest[:NBUF]
    sems = rest[NBUF:2 * NBUF]
    vsems = rest[2 * NBUF:]

    cid = lax.axis_index("c")
    sid = lax.axis_index("s")
    wid = sid * N_CORES + cid
    base = wid * ew
    lane = lax.iota(jnp.int32, LANES)
    rots = [jnp.bitwise_and(lane + r, LANES - 1) for r in (8, 4, 2, 1)]
    places = [jnp.bitwise_and(lane - k, LANES - 1) for k in range(LANES)]

    def rot(x, perm):
      return x.at[perm].get(mode="promise_in_bounds")

    pltpu.sync_copy(src_hbm.at[pl.ds(base, ew)], src_all)
    pltpu.sync_copy(dst_hbm.at[pl.ds(base, ew)], dst_all)

    row0 = jnp.minimum(sid * STAGE_ROWS, n_nodes - STAGE_ROWS)

    def stage_body(blk, carry):
      r0 = row0 + blk * STAGE_BLK
      pltpu.sync_copy(h_hbm.at[pl.ds(r0, STAGE_BLK)], fbuf)

      def pack_row(r, carry2):
        for i in range(WPF):
          a = fbuf[r, pl.ds(2 * i * LANES, LANES)]
          c = fbuf[r, pl.ds((2 * i + 1) * LANES, LANES)]
          w = plsc.bitcast(
              plsc.pack(a, c, format=plsc.PackFormat.INTERLEAVED),
              jnp.float32)
          pbuf[r, pl.ds(i * LANES, LANES)] = w
        return carry2

      lax.fori_loop(0, STAGE_BLK, pack_row, 0)
      pltpu.sync_copy(pbuf, h_sh.at[pl.ds(r0, STAGE_BLK)])
      return carry

    lax.fori_loop(0, STAGE_ROWS // STAGE_BLK, stage_body, 0)
    plsc.subcore_barrier()

    def fire(ch, b):
      iu = src_all.at[pl.ds(ch * CHUNK, CHUNK)]
      iv = dst_all.at[pl.ds(ch * CHUNK, CHUNK)]
      pltpu.async_copy(h_sh.at[iu], rows[b].at[pl.ds(0, CHUNK)], sems[b])
      pltpu.async_copy(h_sh.at[iv], rows[b].at[pl.ds(CHUNK, CHUNK)],
                       vsems[b])

    def wait_gather(ch, b):
      iu = src_all.at[pl.ds(ch * CHUNK, CHUNK)]
      iv = dst_all.at[pl.ds(ch * CHUNK, CHUNK)]
      pltpu.make_async_copy(
          h_sh.at[iu], rows[b].at[pl.ds(0, CHUNK)], sems[b]).wait()
      pltpu.make_async_copy(
          h_sh.at[iv], rows[b].at[pl.ds(CHUNK, CHUNK)], vsems[b]).wait()

    for b in range(NBUF):
      fire(b, b)

    def compute_chunk(ch, b):
      def group_body(g, carry2, b=b):
        acc = jnp.zeros((LANES,), jnp.float32)
        for k in range(LANES):
          e_ = g * LANES + k
          ps = []
          for i in range(WPF):
            uw = plsc.bitcast(rows[b][e_, pl.ds(i * LANES, LANES)],
                              jnp.bfloat16)
            vw = plsc.bitcast(rows[b][CHUNK + e_, pl.ds(i * LANES, LANES)],
                              jnp.bfloat16)
            lo, hi = plsc.unpack(uw * vw, format=plsc.PackFormat.INTERLEAVED)
            ps.append(lo + hi)
          m = (ps[0] + ps[1]) + (ps[2] + ps[3])
          for p in rots:
            m = m + rot(m, p)
          t = m if k == 0 else rot(m, places[k])
          acc = jnp.where(lane == k, t, acc)
        out_all[pl.ds(ch * CHUNK + g * LANES, LANES)] = acc
        return carry2

      lax.fori_loop(0, GROUPS, group_body, 0)

    n_pairs = (n_chunks - 1) // NBUF

    def loop_body(j, carry):
      for b in range(NBUF):
        ch = NBUF * j + b
        wait_gather(ch, b)
        compute_chunk(ch, b)
        fire(jnp.minimum(ch + NBUF, n_chunks - 1), b)
      return carry

    lax.fori_loop(0, n_pairs, loop_body, 0)
    if n_chunks % NBUF:
      tail = n_chunks - 1
      wait_gather(tail, tail % NBUF)
      compute_chunk(tail, tail % NBUF)
      wait_gather(0, (tail + 1) % NBUF)
    else:
      for b in range(NBUF):
        wait_gather(0, b)
    pltpu.sync_copy(out_all, out_hbm.at[pl.ds(base, ew)])

  return score_kernel


def kernel(h, edge_index):
  e = edge_index.shape[1]
  src = edge_index[0].astype(jnp.int32)
  dst = edge_index[1].astype(jnp.int32)
  out = _make_kernel(e, h.shape[0])(h, src, dst)
  return out[:, None]


# speedup vs baseline: 1.3876x; 1.0781x over previous
import functools

import jax
import jax.numpy as jnp
from jax import lax
from jax.experimental import pallas as pl
from jax.experimental.pallas import tpu as pltpu
from jax.experimental.pallas import tpu_sc as plsc

D_FEAT = 128
LANES = 16
N_CORES = 2
N_SUBCORES = 16
N_WORKERS = N_CORES * N_SUBCORES
CHUNK = 80
GROUPS = CHUNK // LANES
D_WORDS = D_FEAT // 2
WPF = D_WORDS // LANES
NBUF = 2
STAGE_ROWS = 640
STAGE_BLK = 32


def _make_kernel(e, n_nodes):
  ew = e // N_WORKERS
  n_chunks = ew // CHUNK
  assert ew % CHUNK == 0 and n_nodes % 8 == 0 and n_nodes >= STAGE_ROWS
  mesh = plsc.VectorSubcoreMesh(core_axis_name="c", subcore_axis_name="s")

  @functools.partial(
      pl.kernel,
      mesh=mesh,
      compiler_params=pltpu.CompilerParams(needs_layout_passes=False,
                                           use_tc_tiling_on_sc=False),
      out_type=jax.ShapeDtypeStruct((e,), jnp.float32),
      scratch_types=[
          pltpu.VMEM_SHARED((n_nodes, D_WORDS), jnp.float32),
          pltpu.VMEM((ew,), jnp.int32),
          pltpu.VMEM((ew,), jnp.int32),
          pltpu.VMEM((ew,), jnp.float32),
          pltpu.VMEM((STAGE_BLK, D_FEAT), jnp.float32),
          pltpu.VMEM((STAGE_BLK, D_WORDS), jnp.float32),
      ] + [pltpu.VMEM((2 * CHUNK, D_WORDS), jnp.float32)] * NBUF
        + [pltpu.SemaphoreType.DMA] * (2 * NBUF),
  )
  def score_kernel(h_hbm, edge_hbm, out_hbm, h_sh,
                   src_all, dst_all, out_all, fbuf, pbuf, *rest):
    rows = rest[:NBUF]
    sems = rest[NBUF:2 * NBUF]
    vsems = rest[2 * NBUF:]

    cid = lax.axis_index("c")
    sid = lax.axis_index("s")
    wid = sid * N_CORES + cid
    base = wid * ew
    lane = lax.iota(jnp.int32, LANES)
    rots = [jnp.bitwise_and(lane + r, LANES - 1) for r in (8, 4, 2, 1)]
    places = [jnp.bitwise_and(lane - k, LANES - 1) for k in range(LANES)]

    def rot(x, perm):
      return x.at[perm].get(mode="promise_in_bounds")

    pltpu.sync_copy(edge_hbm.at[0, pl.ds(base, ew)], src_all)
    pltpu.sync_copy(edge_hbm.at[1, pl.ds(base, ew)], dst_all)

    row0 = jnp.minimum(sid * STAGE_ROWS, n_nodes - STAGE_ROWS)

    def stage_body(blk, carry):
      r0 = row0 + blk * STAGE_BLK
      pltpu.sync_copy(h_hbm.at[pl.ds(r0, STAGE_BLK)], fbuf)

      def pack_row(r, carry2):
        for i in range(WPF):
          a = fbuf[r, pl.ds(2 * i * LANES, LANES)]
          c = fbuf[r, pl.ds((2 * i + 1) * LANES, LANES)]
          w = plsc.bitcast(
              plsc.pack(a, c, format=plsc.PackFormat.INTERLEAVED),
              jnp.float32)
          pbuf[r, pl.ds(i * LANES, LANES)] = w
        return carry2

      lax.fori_loop(0, STAGE_BLK, pack_row, 0)
      pltpu.sync_copy(pbuf, h_sh.at[pl.ds(r0, STAGE_BLK)])
      return carry

    lax.fori_loop(0, STAGE_ROWS // STAGE_BLK, stage_body, 0)
    plsc.subcore_barrier()

    def fire(ch, b):
      iu = src_all.at[pl.ds(ch * CHUNK, CHUNK)]
      iv = dst_all.at[pl.ds(ch * CHUNK, CHUNK)]
      pltpu.async_copy(h_sh.at[iu], rows[b].at[pl.ds(0, CHUNK)], sems[b])
      pltpu.async_copy(h_sh.at[iv], rows[b].at[pl.ds(CHUNK, CHUNK)],
                       vsems[b])

    def wait_gather(ch, b):
      iu = src_all.at[pl.ds(ch * CHUNK, CHUNK)]
      iv = dst_all.at[pl.ds(ch * CHUNK, CHUNK)]
      pltpu.make_async_copy(
          h_sh.at[iu], rows[b].at[pl.ds(0, CHUNK)], sems[b]).wait()
      pltpu.make_async_copy(
          h_sh.at[iv], rows[b].at[pl.ds(CHUNK, CHUNK)], vsems[b]).wait()

    for b in range(NBUF):
      fire(b, b)

    def compute_chunk(ch, b):
      def group_body(g, carry2, b=b):
        acc = jnp.zeros((LANES,), jnp.float32)
        for k in range(LANES):
          e_ = g * LANES + k
          ps = []
          for i in range(WPF):
            uw = plsc.bitcast(rows[b][e_, pl.ds(i * LANES, LANES)],
                              jnp.bfloat16)
            vw = plsc.bitcast(rows[b][CHUNK + e_, pl.ds(i * LANES, LANES)],
                              jnp.bfloat16)
            lo, hi = plsc.unpack(uw * vw, format=plsc.PackFormat.INTERLEAVED)
            ps.append(lo + hi)
          m = (ps[0] + ps[1]) + (ps[2] + ps[3])
          for p in rots:
            m = m + rot(m, p)
          t = m if k == 0 else rot(m, places[k])
          acc = jnp.where(lane == k, t, acc)
        out_all[pl.ds(ch * CHUNK + g * LANES, LANES)] = acc
        return carry2

      lax.fori_loop(0, GROUPS, group_body, 0)

    n_pairs = (n_chunks - 1) // NBUF

    def loop_body(j, carry):
      for b in range(NBUF):
        ch = NBUF * j + b
        wait_gather(ch, b)
        compute_chunk(ch, b)
        fire(jnp.minimum(ch + NBUF, n_chunks - 1), b)
      return carry

    lax.fori_loop(0, n_pairs, loop_body, 0)
    if n_chunks % NBUF:
      tail = n_chunks - 1
      wait_gather(tail, tail % NBUF)
      compute_chunk(tail, tail % NBUF)
      wait_gather(0, (tail + 1) % NBUF)
    else:
      for b in range(NBUF):
        wait_gather(0, b)
    pltpu.sync_copy(out_all, out_hbm.at[pl.ds(base, ew)])

  return score_kernel


def kernel(h, edge_index):
  e = edge_index.shape[1]
  if edge_index.dtype != jnp.int32:
    edge_index = edge_index.astype(jnp.int32)
  out = _make_kernel(e, h.shape[0])(h, edge_index)
  return out[:, None]
